# baseline (speedup 1.0000x reference)
import functools

import jax
import jax.numpy as jnp
from jax import lax
from jax.experimental import pallas as pl
from jax.experimental.pallas import tpu as pltpu

ROWS = 128
COLS = 32768
BR = 64
GRID = ROWS // BR
HALF = COLS // 2


def _tc_body(l_ref, r_ref, o_ref):
    l = l_ref[...]
    r = r_ref[...]
    vl = jnp.min(l, axis=1)
    il = jnp.argmin(l, axis=1).astype(jnp.int32)
    vr = jnp.min(r, axis=1)
    ir = jnp.argmin(r, axis=1).astype(jnp.int32)
    pred = vr < vl
    idx = jnp.where(pred, ir + HALF, il)
    o_ref[...] = idx.reshape(1, 1, BR)


@functools.partial(jax.jit)
def kernel(x):
    out = pl.pallas_call(
        _tc_body,
        out_shape=jax.ShapeDtypeStruct((GRID, 1, BR), jnp.int32),
        grid=(GRID,),
        in_specs=[
            pl.BlockSpec((BR, HALF), lambda i: (i, 0)),
            pl.BlockSpec((BR, HALF), lambda i: (i, 1)),
        ],
        out_specs=pl.BlockSpec((1, 1, BR), lambda i: (i, 0, 0)),
    )(x, x)
    return out.reshape(ROWS)
